# Initial kernel scaffold; baseline (speedup 1.0000x reference)
import jax
import jax.numpy as jnp
from jax.experimental import pallas as pl


def kernel(x, edge_index, W_self_0, W_neigh_0, b_0, W_self_1, W_neigh_1, b_1, W_self_2, W_neigh_2, b_2):
    raise NotImplementedError("write your pallas kernel here")


# speedup vs baseline: 4.1351x; 4.1351x over previous
import functools

import jax
import jax.numpy as jnp
from jax import lax
from jax.experimental import pallas as pl
from jax.experimental.pallas import tpu as pltpu
from jax.experimental.pallas import tpu_sc as plsc

N_NODES = 10000
N_PAD = 10240
N_SUBCORES = 16
EDGE_BATCH = 128
ROWS_PER_TILE = N_PAD // N_SUBCORES


def _make_sc_agg(n_idx_rows, fh, with_deg):
    rpt = n_idx_rows // N_SUBCORES
    nzc = ROWS_PER_TILE // EDGE_BATCH
    ic = 32
    n_ic = rpt // ic

    out_type = [jax.ShapeDtypeStruct((2, N_PAD, fh), jnp.float32)]
    scratch = [
        pltpu.VMEM((ic, EDGE_BATCH), jnp.int32),
        pltpu.VMEM((ic, EDGE_BATCH), jnp.int32),
        pltpu.VMEM((EDGE_BATCH, fh), jnp.float32),
        pltpu.VMEM_SHARED((N_PAD, fh), jnp.float32),
        pltpu.SemaphoreType.DMA,
    ]
    if with_deg:
        out_type.append(jax.ShapeDtypeStruct((N_PAD,), jnp.float32))
        scratch += [
            pltpu.VMEM((N_PAD,), jnp.float32),
            pltpu.VMEM_SHARED((N_SUBCORES, N_PAD), jnp.float32),
            pltpu.VMEM((N_SUBCORES, ROWS_PER_TILE), jnp.float32),
            pltpu.VMEM((ROWS_PER_TILE,), jnp.float32),
        ]

    mesh = plsc.VectorSubcoreMesh(core_axis_name="c", subcore_axis_name="s")

    def body(*refs):
        if with_deg:
            (h0, h1, srcr, dstr, z2dr, z1dr, aggr, degr,
             srcv, dstv, rowsv, aggsh, sem, hist, grid, red, degv) = refs
        else:
            (h0, h1, srcr, dstr, z2dr, aggr,
             srcv, dstv, rowsv, aggsh, sem) = refs
        c = lax.axis_index("c")
        s = lax.axis_index("s")
        row0 = s * ROWS_PER_TILE
        for i in range(nzc):
            pltpu.sync_copy(z2dr, aggsh.at[pl.ds(row0 + i * EDGE_BATCH, EDGE_BATCH)])
        if with_deg:
            pltpu.sync_copy(z1dr, hist)
        ones = jnp.full((16,), 1.0, jnp.float32)
        r0 = s * rpt
        plsc.subcore_barrier()

        def outer(ci, carry):
            pltpu.sync_copy(srcr.at[pl.ds(r0 + ci * ic, ic)], srcv)
            pltpu.sync_copy(dstr.at[pl.ds(r0 + ci * ic, ic)], dstv)

            def step(j, carry2):
                @pl.when(c == 0)
                def _():
                    pltpu.async_copy(h0.at[srcv.at[j]], rowsv, sem).wait()

                @pl.when(c != 0)
                def _():
                    pltpu.async_copy(h1.at[srcv.at[j]], rowsv, sem).wait()

                pltpu.sync_copy(rowsv, aggsh.at[dstv.at[j]], add=True)
                if with_deg:
                    for k in range(EDGE_BATCH // 16):
                        idx = dstv[j, pl.ds(k * 16, 16)]
                        plsc.addupdate_scatter(hist, (idx,), ones)
                return carry2

            lax.fori_loop(0, ic, step, 0)
            return carry

        lax.fori_loop(0, n_ic, outer, 0)
        plsc.subcore_barrier()
        pltpu.sync_copy(
            aggsh.at[pl.ds(row0, ROWS_PER_TILE)],
            aggr.at[c, pl.ds(row0, ROWS_PER_TILE)],
        )

        if with_deg:
            pltpu.sync_copy(hist, grid.at[s])
            plsc.subcore_barrier()
            pltpu.sync_copy(grid.at[:, pl.ds(row0, ROWS_PER_TILE)], red)

            def rstep(k, carry):
                acc = jnp.zeros((16,), jnp.float32)
                for r in range(N_SUBCORES):
                    acc = acc + red[r, pl.ds(k * 16, 16)]
                degv[pl.ds(k * 16, 16)] = acc
                return carry

            lax.fori_loop(0, ROWS_PER_TILE // 16, rstep, 0)

            @pl.when(c == 0)
            def _():
                pltpu.sync_copy(degv, degr.at[pl.ds(row0, ROWS_PER_TILE)])

    return pl.kernel(
        body, out_type=out_type, mesh=mesh, scratch_types=scratch,
        compiler_params=pltpu.CompilerParams(
            needs_layout_passes=False, use_tc_tiling_on_sc=False))


_HI = jax.lax.Precision.HIGHEST


def _dot(a, b):
    return jnp.dot(a, b, preferred_element_type=jnp.float32, precision=_HI)


def _tc_layer01(h_a, h_b, agg, deg2d, ws, wn, b2d, wnx=None):
    _, np_, fh = agg.shape
    din = 2 * fh
    dh = din // 2
    dout = ws.shape[1]
    bm = 256
    nblk = np_ // bm

    def body(ha, hb, ag, dg, wsr, wnr, br, *rest):
        inv = 1.0 / jnp.maximum(dg[...], 1.0)
        acc = _dot(ha[...], wsr[pl.ds(0, dh), :])
        acc += _dot(hb[...], wsr[pl.ds(dh, dh), :])
        acc += _dot(ag[0] * inv, wnr[pl.ds(0, fh), :])
        acc += _dot(ag[1] * inv, wnr[pl.ds(fh, fh), :])
        acc += br[...]
        acc = jnp.maximum(acc, 0.0)
        if wnx is None:
            oa, ob = rest
            oa[...] = acc[:, : dout // 2]
            ob[...] = acc[:, dout // 2:]
        else:
            wxr, oa, ob, xa, xb = rest
            oa[...] = acc[:, : dout // 2]
            ob[...] = acc[:, dout // 2:]
            nxt = _dot(acc, wxr[...])
            dx = nxt.shape[1] // 2
            xa[...] = nxt[:, :dx]
            xb[...] = nxt[:, dx:]

    in_specs = [
        pl.BlockSpec((bm, dh), lambda i: (i, 0)),
        pl.BlockSpec((bm, dh), lambda i: (i, 0)),
        pl.BlockSpec((2, bm, fh), lambda i: (0, i, 0)),
        pl.BlockSpec((bm, 1), lambda i: (i, 0)),
        pl.BlockSpec((din, dout), lambda i: (0, 0)),
        pl.BlockSpec((din, dout), lambda i: (0, 0)),
        pl.BlockSpec((1, dout), lambda i: (0, 0)),
    ]
    out_shape = [
        jax.ShapeDtypeStruct((np_, dout // 2), jnp.float32),
        jax.ShapeDtypeStruct((np_, dout // 2), jnp.float32),
    ]
    out_specs = [
        pl.BlockSpec((bm, dout // 2), lambda i: (i, 0)),
        pl.BlockSpec((bm, dout // 2), lambda i: (i, 0)),
    ]
    args = [h_a, h_b, agg, deg2d, ws, wn, b2d]
    if wnx is not None:
        dx = wnx.shape[1]
        in_specs.append(pl.BlockSpec((dout, dx), lambda i: (0, 0)))
        out_shape += [
            jax.ShapeDtypeStruct((np_, dx // 2), jnp.float32),
            jax.ShapeDtypeStruct((np_, dx // 2), jnp.float32),
        ]
        out_specs += [
            pl.BlockSpec((bm, dx // 2), lambda i: (i, 0)),
            pl.BlockSpec((bm, dx // 2), lambda i: (i, 0)),
        ]
        args.append(wnx)
    return pl.pallas_call(
        body,
        grid=(nblk,),
        in_specs=in_specs,
        out_specs=out_specs,
        out_shape=out_shape,
    )(*args)


def _tc_final(h_a, h_b, agg, deg2d, ws, b2d):
    np_, din = h_a.shape[0], h_a.shape[1] * 2
    dh = din // 2
    dout = ws.shape[1]
    fh = dout // 2
    bm = 256
    nblk = np_ // bm

    def body(ha, hb, ag, dg, wsr, br, o):
        inv = 1.0 / jnp.maximum(dg[...], 1.0)
        acc = _dot(ha[...], wsr[pl.ds(0, dh), :])
        acc += _dot(hb[...], wsr[pl.ds(dh, dh), :])
        acc += jnp.concatenate([ag[0], ag[1]], axis=1) * inv
        acc += br[...]
        o[...] = acc

    return pl.pallas_call(
        body,
        grid=(nblk,),
        in_specs=[
            pl.BlockSpec((bm, dh), lambda i: (i, 0)),
            pl.BlockSpec((bm, dh), lambda i: (i, 0)),
            pl.BlockSpec((2, bm, fh), lambda i: (0, i, 0)),
            pl.BlockSpec((bm, 1), lambda i: (i, 0)),
            pl.BlockSpec((din, dout), lambda i: (0, 0)),
            pl.BlockSpec((1, dout), lambda i: (0, 0)),
        ],
        out_specs=pl.BlockSpec((bm, dout), lambda i: (i, 0)),
        out_shape=jax.ShapeDtypeStruct((np_, dout), jnp.float32),
    )(h_a, h_b, agg, deg2d, ws, b2d)


def kernel(x, edge_index, W_self_0, W_neigh_0, b_0, W_self_1, W_neigh_1, b_1,
           W_self_2, W_neigh_2, b_2):
    n, fin = x.shape
    e = edge_index.shape[1]
    quant = N_SUBCORES * EDGE_BATCH * 8
    ep = (e + quant - 1) // quant * quant
    src = jnp.concatenate(
        [edge_index[0], jnp.zeros((ep - e,), jnp.int32)]).reshape(-1, EDGE_BATCH)
    dst = jnp.concatenate(
        [edge_index[1], jnp.full((ep - e,), N_NODES, jnp.int32)]).reshape(-1, EDGE_BATCH)
    n_idx_rows = ep // EDGE_BATCH

    z1d = jnp.zeros((N_PAD,), jnp.float32)
    xp = jnp.pad(x, ((0, N_PAD - n), (0, 0)))
    x0 = xp[:, : fin // 2]
    x1 = xp[:, fin // 2:]

    agg0, deg = _make_sc_agg(n_idx_rows, fin // 2, True)(
        x0, x1, src, dst, jnp.zeros((EDGE_BATCH, fin // 2), jnp.float32), z1d)
    deg2d = deg[:, None]
    b0 = b_0[None, :]
    h1a, h1b = _tc_layer01(x0, x1, agg0, deg2d, W_self_0, W_neigh_0, b0)

    d1 = W_self_1.shape[1]
    (agg1,) = _make_sc_agg(n_idx_rows, d1 // 2, False)(
        h1a, h1b, src, dst, jnp.zeros((EDGE_BATCH, d1 // 2), jnp.float32))
    d2p = 64
    wn2p = jnp.pad(W_neigh_2, ((0, 0), (0, d2p - W_neigh_2.shape[1])))
    h2a, h2b, hwa, hwb = _tc_layer01(
        h1a, h1b, agg1, deg2d, W_self_1, W_neigh_1, b_1[None, :], wnx=wn2p)

    (agg2,) = _make_sc_agg(n_idx_rows, d2p // 2, False)(
        hwa, hwb, src, dst, jnp.zeros((EDGE_BATCH, d2p // 2), jnp.float32))
    ws2p = jnp.pad(W_self_2, ((0, 0), (0, d2p - W_self_2.shape[1])))
    b2p = jnp.pad(b_2, (0, d2p - b_2.shape[0]))[None, :]
    out = _tc_final(h2a, h2b, agg2, deg2d, ws2p, b2p)
    return out[:N_NODES, : W_self_2.shape[1]]


# speedup vs baseline: 5.1658x; 1.2493x over previous
import functools

import jax
import jax.numpy as jnp
from jax import lax
from jax.experimental import pallas as pl
from jax.experimental.pallas import tpu as pltpu
from jax.experimental.pallas import tpu_sc as plsc

N_NODES = 10000
N_PAD = 10240
N_SUBCORES = 16
EDGE_BATCH = 128
ROWS_PER_TILE = N_PAD // N_SUBCORES


def _make_sc_agg(n_idx_rows, fh, with_deg):
    rpt = n_idx_rows // N_SUBCORES
    nzc = ROWS_PER_TILE // EDGE_BATCH
    ic = 32
    n_ic = rpt // ic

    out_type = [jax.ShapeDtypeStruct((2, N_PAD, fh), jnp.float32)]
    scratch = [
        pltpu.VMEM((ic, EDGE_BATCH), jnp.int32),
        pltpu.VMEM((ic, EDGE_BATCH), jnp.int32),
        pltpu.VMEM((EDGE_BATCH, fh), jnp.float32),
        pltpu.VMEM((EDGE_BATCH, fh), jnp.float32),
        pltpu.VMEM_SHARED((N_PAD, fh), jnp.float32),
        pltpu.SemaphoreType.DMA,
        pltpu.SemaphoreType.DMA,
    ]
    if with_deg:
        out_type.append(jax.ShapeDtypeStruct((N_PAD,), jnp.float32))
        scratch += [
            pltpu.VMEM((N_PAD,), jnp.float32),
            pltpu.VMEM_SHARED((N_SUBCORES, N_PAD), jnp.float32),
            pltpu.VMEM((N_SUBCORES, ROWS_PER_TILE), jnp.float32),
            pltpu.VMEM((ROWS_PER_TILE,), jnp.float32),
        ]

    mesh = plsc.VectorSubcoreMesh(core_axis_name="c", subcore_axis_name="s")

    nbuf = 2

    def body(*refs):
        if with_deg:
            (h0, h1, srcr, dstr, z2dr, z1dr, aggr, degr,
             srcv, dstv, rows0, rows1, aggsh, sem0, sem1,
             hist, grid, red, degv) = refs
        else:
            (h0, h1, srcr, dstr, z2dr, aggr,
             srcv, dstv, rows0, rows1, aggsh, sem0, sem1) = refs
        c = lax.axis_index("c")
        s = lax.axis_index("s")
        row0 = s * ROWS_PER_TILE
        for i in range(nzc):
            pltpu.sync_copy(z2dr, aggsh.at[pl.ds(row0 + i * EDGE_BATCH, EDGE_BATCH)])
        if with_deg:
            pltpu.sync_copy(z1dr, hist)
        ones = jnp.full((16,), 1.0, jnp.float32)
        r0 = s * rpt
        bufs = (rows0, rows1)
        sems = (sem0, sem1)
        plsc.subcore_barrier()

        def pipeline(h):
            def outer(ci, carry):
                pltpu.sync_copy(srcr.at[pl.ds(r0 + ci * ic, ic)], srcv)
                pltpu.sync_copy(dstr.at[pl.ds(r0 + ci * ic, ic)], dstv)
                descs = [
                    pltpu.async_copy(h.at[srcv.at[b]], bufs[b], sems[b])
                    for b in range(nbuf)
                ]
                for j in range(ic):
                    b = j % nbuf
                    descs[b].wait()
                    pltpu.sync_copy(bufs[b], aggsh.at[dstv.at[j]], add=True)
                    if with_deg:
                        for k in range(EDGE_BATCH // 16):
                            idx = dstv[j, pl.ds(k * 16, 16)]
                            plsc.addupdate_scatter(hist, (idx,), ones)
                    if j + nbuf < ic:
                        descs[b] = pltpu.async_copy(
                            h.at[srcv.at[j + nbuf]], bufs[b], sems[b])
                return carry

            lax.fori_loop(0, n_ic, outer, 0)

        @pl.when(c == 0)
        def _():
            pipeline(h0)

        @pl.when(c != 0)
        def _():
            pipeline(h1)

        plsc.subcore_barrier()
        pltpu.sync_copy(
            aggsh.at[pl.ds(row0, ROWS_PER_TILE)],
            aggr.at[c, pl.ds(row0, ROWS_PER_TILE)],
        )

        if with_deg:
            pltpu.sync_copy(hist, grid.at[s])
            plsc.subcore_barrier()
            pltpu.sync_copy(grid.at[:, pl.ds(row0, ROWS_PER_TILE)], red)

            def rstep(k, carry):
                acc = jnp.zeros((16,), jnp.float32)
                for r in range(N_SUBCORES):
                    acc = acc + red[r, pl.ds(k * 16, 16)]
                degv[pl.ds(k * 16, 16)] = acc
                return carry

            lax.fori_loop(0, ROWS_PER_TILE // 16, rstep, 0)

            @pl.when(c == 0)
            def _():
                pltpu.sync_copy(degv, degr.at[pl.ds(row0, ROWS_PER_TILE)])

    return pl.kernel(
        body, out_type=out_type, mesh=mesh, scratch_types=scratch,
        compiler_params=pltpu.CompilerParams(
            needs_layout_passes=False, use_tc_tiling_on_sc=False))


_HI = jax.lax.Precision.HIGHEST


def _dot(a, b):
    return jnp.dot(a, b, preferred_element_type=jnp.float32, precision=_HI)


def _tc_layer01(h_a, h_b, agg, deg2d, ws, wn, b2d, wnx=None):
    _, np_, fh = agg.shape
    din = 2 * fh
    dh = din // 2
    dout = ws.shape[1]
    bm = 256
    nblk = np_ // bm

    def body(ha, hb, ag, dg, wsr, wnr, br, *rest):
        inv = 1.0 / jnp.maximum(dg[...], 1.0)
        acc = _dot(ha[...], wsr[pl.ds(0, dh), :])
        acc += _dot(hb[...], wsr[pl.ds(dh, dh), :])
        acc += _dot(ag[0] * inv, wnr[pl.ds(0, fh), :])
        acc += _dot(ag[1] * inv, wnr[pl.ds(fh, fh), :])
        acc += br[...]
        acc = jnp.maximum(acc, 0.0)
        if wnx is None:
            oa, ob = rest
            oa[...] = acc[:, : dout // 2]
            ob[...] = acc[:, dout // 2:]
        else:
            wxr, oa, ob, xa, xb = rest
            oa[...] = acc[:, : dout // 2]
            ob[...] = acc[:, dout // 2:]
            nxt = _dot(acc, wxr[...])
            dx = nxt.shape[1] // 2
            xa[...] = nxt[:, :dx]
            xb[...] = nxt[:, dx:]

    in_specs = [
        pl.BlockSpec((bm, dh), lambda i: (i, 0)),
        pl.BlockSpec((bm, dh), lambda i: (i, 0)),
        pl.BlockSpec((2, bm, fh), lambda i: (0, i, 0)),
        pl.BlockSpec((bm, 1), lambda i: (i, 0)),
        pl.BlockSpec((din, dout), lambda i: (0, 0)),
        pl.BlockSpec((din, dout), lambda i: (0, 0)),
        pl.BlockSpec((1, dout), lambda i: (0, 0)),
    ]
    out_shape = [
        jax.ShapeDtypeStruct((np_, dout // 2), jnp.float32),
        jax.ShapeDtypeStruct((np_, dout // 2), jnp.float32),
    ]
    out_specs = [
        pl.BlockSpec((bm, dout // 2), lambda i: (i, 0)),
        pl.BlockSpec((bm, dout // 2), lambda i: (i, 0)),
    ]
    args = [h_a, h_b, agg, deg2d, ws, wn, b2d]
    if wnx is not None:
        dx = wnx.shape[1]
        in_specs.append(pl.BlockSpec((dout, dx), lambda i: (0, 0)))
        out_shape += [
            jax.ShapeDtypeStruct((np_, dx // 2), jnp.float32),
            jax.ShapeDtypeStruct((np_, dx // 2), jnp.float32),
        ]
        out_specs += [
            pl.BlockSpec((bm, dx // 2), lambda i: (i, 0)),
            pl.BlockSpec((bm, dx // 2), lambda i: (i, 0)),
        ]
        args.append(wnx)
    return pl.pallas_call(
        body,
        grid=(nblk,),
        in_specs=in_specs,
        out_specs=out_specs,
        out_shape=out_shape,
    )(*args)


def _tc_final(h_a, h_b, agg, deg2d, ws, b2d):
    np_, din = h_a.shape[0], h_a.shape[1] * 2
    dh = din // 2
    dout = ws.shape[1]
    fh = dout // 2
    bm = 256
    nblk = np_ // bm

    def body(ha, hb, ag, dg, wsr, br, o):
        inv = 1.0 / jnp.maximum(dg[...], 1.0)
        acc = _dot(ha[...], wsr[pl.ds(0, dh), :])
        acc += _dot(hb[...], wsr[pl.ds(dh, dh), :])
        acc += jnp.concatenate([ag[0], ag[1]], axis=1) * inv
        acc += br[...]
        o[...] = acc

    return pl.pallas_call(
        body,
        grid=(nblk,),
        in_specs=[
            pl.BlockSpec((bm, dh), lambda i: (i, 0)),
            pl.BlockSpec((bm, dh), lambda i: (i, 0)),
            pl.BlockSpec((2, bm, fh), lambda i: (0, i, 0)),
            pl.BlockSpec((bm, 1), lambda i: (i, 0)),
            pl.BlockSpec((din, dout), lambda i: (0, 0)),
            pl.BlockSpec((1, dout), lambda i: (0, 0)),
        ],
        out_specs=pl.BlockSpec((bm, dout), lambda i: (i, 0)),
        out_shape=jax.ShapeDtypeStruct((np_, dout), jnp.float32),
    )(h_a, h_b, agg, deg2d, ws, b2d)


def kernel(x, edge_index, W_self_0, W_neigh_0, b_0, W_self_1, W_neigh_1, b_1,
           W_self_2, W_neigh_2, b_2):
    n, fin = x.shape
    e = edge_index.shape[1]
    quant = N_SUBCORES * EDGE_BATCH * 8
    ep = (e + quant - 1) // quant * quant
    src = jnp.concatenate(
        [edge_index[0], jnp.zeros((ep - e,), jnp.int32)]).reshape(-1, EDGE_BATCH)
    dst = jnp.concatenate(
        [edge_index[1], jnp.full((ep - e,), N_NODES, jnp.int32)]).reshape(-1, EDGE_BATCH)
    n_idx_rows = ep // EDGE_BATCH

    z1d = jnp.zeros((N_PAD,), jnp.float32)
    xp = jnp.pad(x, ((0, N_PAD - n), (0, 0)))
    x0 = xp[:, : fin // 2]
    x1 = xp[:, fin // 2:]

    agg0, deg = _make_sc_agg(n_idx_rows, fin // 2, True)(
        x0, x1, src, dst, jnp.zeros((EDGE_BATCH, fin // 2), jnp.float32), z1d)
    deg2d = deg[:, None]
    b0 = b_0[None, :]
    h1a, h1b = _tc_layer01(x0, x1, agg0, deg2d, W_self_0, W_neigh_0, b0)

    d1 = W_self_1.shape[1]
    (agg1,) = _make_sc_agg(n_idx_rows, d1 // 2, False)(
        h1a, h1b, src, dst, jnp.zeros((EDGE_BATCH, d1 // 2), jnp.float32))
    d2p = 64
    wn2p = jnp.pad(W_neigh_2, ((0, 0), (0, d2p - W_neigh_2.shape[1])))
    h2a, h2b, hwa, hwb = _tc_layer01(
        h1a, h1b, agg1, deg2d, W_self_1, W_neigh_1, b_1[None, :], wnx=wn2p)

    (agg2,) = _make_sc_agg(n_idx_rows, d2p // 2, False)(
        hwa, hwb, src, dst, jnp.zeros((EDGE_BATCH, d2p // 2), jnp.float32))
    ws2p = jnp.pad(W_self_2, ((0, 0), (0, d2p - W_self_2.shape[1])))
    b2p = jnp.pad(b_2, (0, d2p - b_2.shape[0]))[None, :]
    out = _tc_final(h2a, h2b, agg2, deg2d, ws2p, b2p)
    return out[:N_NODES, : W_self_2.shape[1]]
